# baseline (speedup 1.0000x reference)
import functools

import jax
import jax.numpy as jnp
from jax import lax
from jax.experimental import pallas as pl
from jax.experimental.pallas import tpu as pltpu
from jax.experimental.pallas import tpu_sc as plsc

NUM_ACTIVATIONS = 65536
D_MODEL = 1024
K = 512

NC = 2
NS = 16
L = 16
DH = D_MODEL // NC
KPS = K // NS
RG = 8
NG = KPS // RG
CHUNKS = DH // L

_DN = lax.GatherDimensionNumbers(
    offset_dims=(), collapsed_slice_dims=(0,), start_index_map=(0,))


def _splat(w_reg, i):
  lane = jnp.full((L, 1), i, dtype=jnp.int32)
  return lax.gather(w_reg, lane, _DN, (1,),
                    mode=lax.GatherScatterMode.PROMISE_IN_BOUNDS)


def _tree_sum(terms):
  while len(terms) > 1:
    nxt = [a + b for a, b in zip(terms[::2], terms[1::2])]
    if len(terms) % 2:
      nxt.append(terms[-1])
    terms = nxt
  return terms[0]


def _sc_func_vec(top_indices, weight, W_dec, b_dec, zero):
  mesh = plsc.VectorSubcoreMesh(core_axis_name="c", subcore_axis_name="s")

  @functools.partial(
      pl.kernel,
      out_type=jax.ShapeDtypeStruct((1, D_MODEL), jnp.float32),
      mesh=mesh,
      scratch_types=[
          pltpu.VMEM((KPS,), jnp.int32),
          pltpu.VMEM((KPS,), jnp.float32),
          [pltpu.VMEM((RG, DH), jnp.float32) for _ in range(NG)],
          pltpu.VMEM((1, DH), jnp.float32),
          pltpu.VMEM((1,), jnp.int32),
          pltpu.VMEM_SHARED((1, DH), jnp.float32),
          pltpu.SemaphoreType.DMA,
          pltpu.SemaphoreType.DMA,
          [pltpu.SemaphoreType.DMA for _ in range(NG)],
          pltpu.SemaphoreType.DMA,
      ],
  )
  def body(idx_hbm, w_hbm, wdec_hbm, bdec_hbm, zero_hbm, out_hbm, idx_v, wv,
           rows_v, acc_v, zero_v, shared_acc, sem_w, sem_z, sem_r, sem_b):
    cid = lax.axis_index("c")
    sid = lax.axis_index("s")
    dbase = cid * DH

    bdec_cp = None

    @pl.when(sid == 0)
    def _():
      pltpu.async_copy(bdec_hbm.at[pl.ds(0, 1), pl.ds(dbase, DH)],
                       shared_acc, sem_b).wait()

    pltpu.sync_copy(idx_hbm.at[pl.ds(sid * KPS, KPS)], idx_v)
    cp_w = pltpu.async_copy(w_hbm.at[idx_v], wv, sem_w)
    cp_r = [
        pltpu.async_copy(
            wdec_hbm.at[idx_v.at[pl.ds(g * RG, RG)], pl.ds(dbase, DH)],
            rows_v[g], sem_r[g])
        for g in range(NG)
    ]
    cp_z = pltpu.async_copy(zero_hbm, zero_v, sem_z)

    cp_w.wait()
    w_regs = [jnp.maximum(wv[pl.ds(0, L)], 0.0),
              jnp.maximum(wv[pl.ds(L, L)], 0.0)]

    for g in range(NG):
      bws = [_splat(w_regs[(g * RG) // L], (g * RG) % L + i)
             for i in range(RG)]
      cp_r[g].wait()
      rg_v = rows_v[g]

      if g == 0:
        def group_body(c, carry, bws=bws, rg_v=rg_v):
          sl = pl.ds(c * L, L)
          acc_v[0, sl] = _tree_sum([bws[i] * rg_v[i, sl] for i in range(RG)])
          return carry
      else:
        def group_body(c, carry, bws=bws, rg_v=rg_v):
          sl = pl.ds(c * L, L)
          acc_v[0, sl] = acc_v[0, sl] + _tree_sum(
              [bws[i] * rg_v[i, sl] for i in range(RG)])
          return carry

      lax.fori_loop(0, CHUNKS, group_body, 0)

    plsc.subcore_barrier()
    cp_z.wait()
    pltpu.sync_copy(acc_v, shared_acc.at[zero_v], add=True)
    plsc.subcore_barrier()

    @pl.when(sid == 0)
    def _():
      pltpu.sync_copy(shared_acc, acc_v)
      pltpu.sync_copy(acc_v, out_hbm.at[pl.ds(0, 1), pl.ds(dbase, DH)])

  return body(top_indices, weight, W_dec, b_dec, zero)


def kernel(top_indices, weight, W_dec, b_dec):
  idx = top_indices.astype(jnp.int32)
  zero = jnp.zeros((1,), jnp.int32)
  return _sc_func_vec(idx, weight, W_dec, b_dec.reshape(1, D_MODEL), zero)


# speedup vs baseline: 1.0313x; 1.0313x over previous
import functools

import jax
import jax.numpy as jnp
from jax import lax
from jax.experimental import pallas as pl
from jax.experimental.pallas import tpu as pltpu
from jax.experimental.pallas import tpu_sc as plsc

NUM_ACTIVATIONS = 65536
D_MODEL = 1024
K = 512

NC = 2
NS = 16
L = 16
DH = D_MODEL // NC
KPS = K // NS
CHUNKS = DH // L
UNROLL = 2

_DN = lax.GatherDimensionNumbers(
    offset_dims=(), collapsed_slice_dims=(0,), start_index_map=(0,))


def _splat(w_reg, i):
  lane = jnp.full((L, 1), i, dtype=jnp.int32)
  return lax.gather(w_reg, lane, _DN, (1,),
                    mode=lax.GatherScatterMode.PROMISE_IN_BOUNDS)


def _tree_sum(terms):
  while len(terms) > 1:
    nxt = [a + b for a, b in zip(terms[::2], terms[1::2])]
    if len(terms) % 2:
      nxt.append(terms[-1])
    terms = nxt
  return terms[0]


def _sc_func_vec(top_indices, weight, W_dec, b_dec, zero):
  mesh = plsc.VectorSubcoreMesh(core_axis_name="c", subcore_axis_name="s")

  @functools.partial(
      pl.kernel,
      out_type=jax.ShapeDtypeStruct((1, D_MODEL), jnp.float32),
      mesh=mesh,
      scratch_types=[
          pltpu.VMEM((KPS,), jnp.int32),
          pltpu.VMEM((KPS,), jnp.float32),
          pltpu.VMEM((KPS, DH), jnp.float32),
          pltpu.VMEM((1, DH), jnp.float32),
          pltpu.VMEM((1,), jnp.int32),
          pltpu.VMEM_SHARED((1, DH), jnp.float32),
          pltpu.SemaphoreType.DMA,
          pltpu.SemaphoreType.DMA,
          pltpu.SemaphoreType.DMA,
          pltpu.SemaphoreType.DMA,
      ],
  )
  def body(idx_hbm, w_hbm, wdec_hbm, bdec_hbm, zero_hbm, out_hbm, idx_v, wv,
           rows_v, acc_v, zero_v, shared_acc, sem_w, sem_z, sem_r, sem_b):
    cid = lax.axis_index("c")
    sid = lax.axis_index("s")
    dbase = cid * DH

    @pl.when(sid == 0)
    def _():
      pltpu.async_copy(bdec_hbm.at[pl.ds(0, 1), pl.ds(dbase, DH)],
                       shared_acc, sem_b).wait()

    pltpu.sync_copy(idx_hbm.at[pl.ds(sid * KPS, KPS)], idx_v)
    cp_w = pltpu.async_copy(w_hbm.at[idx_v], wv, sem_w)
    cp_r = pltpu.async_copy(wdec_hbm.at[idx_v, pl.ds(dbase, DH)], rows_v,
                            sem_r)
    cp_z = pltpu.async_copy(zero_hbm, zero_v, sem_z)

    cp_w.wait()
    bws = []
    for h in range(KPS // L):
      w_reg = jnp.maximum(wv[pl.ds(h * L, L)], 0.0)
      bws.extend(_splat(w_reg, i) for i in range(L))

    cp_r.wait()

    def chunk_body(c, carry):
      for u in range(UNROLL):
        sl = pl.ds((c * UNROLL + u) * L, L)
        acc_v[0, sl] = _tree_sum(
            [bws[i] * rows_v[i, sl] for i in range(KPS)])
      return carry

    lax.fori_loop(0, CHUNKS // UNROLL, chunk_body, 0)

    plsc.subcore_barrier()
    cp_z.wait()
    pltpu.sync_copy(acc_v, shared_acc.at[zero_v], add=True)
    plsc.subcore_barrier()

    @pl.when(sid == 0)
    def _():
      pltpu.sync_copy(shared_acc, acc_v)
      pltpu.sync_copy(acc_v, out_hbm.at[pl.ds(0, 1), pl.ds(dbase, DH)])

  return body(top_indices, weight, W_dec, b_dec, zero)


def kernel(top_indices, weight, W_dec, b_dec):
  idx = top_indices.astype(jnp.int32)
  zero = jnp.zeros((1,), jnp.int32)
  return _sc_func_vec(idx, weight, W_dec, b_dec.reshape(1, D_MODEL), zero)
